# Initial kernel scaffold; baseline (speedup 1.0000x reference)
import jax
import jax.numpy as jnp
from jax.experimental import pallas as pl


def kernel(pc1, features):
    raise NotImplementedError("write your pallas kernel here")


# speedup vs baseline: 11.1198x; 11.1198x over previous
import functools

import jax
import jax.numpy as jnp
from jax import lax
from jax.experimental import pallas as pl
from jax.experimental.pallas import tpu as pltpu
from jax.experimental.pallas import tpu_sc as plsc

_S = 512
_NF = _S // 3
_NCELL = _NF * _NF
_DUMP = _NCELL
_GPAD = 28928
_QR = _GPAD // 4
_CH = 128
_NB = 784
_CPT = _NB // 4
_NPAD = _NB * _CH


def _cells_body(pc_ref, cell_ref):
    pcm = pc_ref[0]
    rr = lax.broadcasted_iota(jnp.int32, (3, 3), 0)
    cc = lax.broadcasted_iota(jnp.int32, (3, 3), 1)
    eye = (rr == cc).astype(jnp.float32)
    emat = (3.0 * eye - 1.0) / jnp.sqrt(jnp.float32(6.0))
    ele = jnp.dot(emat, pcm, preferred_element_type=jnp.float32)
    e0 = ele[0:1, :]
    e1 = ele[1:2, :]
    e2 = ele[2:3, :]

    g0 = jnp.round(e0 / 3.0)
    g1 = jnp.round(e1 / 3.0)
    g2 = jnp.round(e2 / 3.0)
    x0 = e0 - g0 * 3.0
    x1 = e1 - g1 * 3.0
    x2 = e2 - g2 * 3.0

    i32 = jnp.int32
    r0 = (x1 > x0).astype(i32) + (x2 > x0).astype(i32)
    r1 = (x0 >= x1).astype(i32) + (x2 > x1).astype(i32)
    r2 = (x0 >= x2).astype(i32) + (x1 >= x2).astype(i32)

    rs = g0 + g1 + g2
    gt = rs > 0.0
    lt = rs < 0.0
    sign = jnp.where(gt, jnp.float32(-1.0), 0.0) + jnp.where(lt, jnp.float32(1.0), 0.0)

    def adjust(rk, g):
        rkf = rk.astype(jnp.float32)
        cond = ((rkf >= 3.0 - rs) & gt) | ((rkf < -rs) & lt)
        adj = sign * cond.astype(jnp.float32)
        g_new = g + adj
        rk_new = rk + (3.0 * adj).astype(i32) + rs.astype(i32)
        return rk_new, g_new

    r0, g0 = adjust(r0, g0)
    r1, g1 = adjust(r1, g1)
    r2, g2 = adjust(r2, g2)

    c0 = (g0 * 3.0).astype(i32)
    c1 = (g1 * 3.0).astype(i32)
    rc0 = jnp.clip(r0, 0, 2)
    rc1 = jnp.clip(r1, 0, 2)

    o0 = jnp.min(c0 - rc0)
    o1 = jnp.min(c1 - rc1)
    pk0 = jnp.mod(-o0, 3)
    pk1 = jnp.mod(-o1, 3)

    u0 = c0 - o0
    u1 = c1 - o1
    f = u0 * _S + u1
    valid = (f >= 0) & (f < _S * _S)
    fr = f >> 9
    fc = f & (_S - 1)
    d0 = fr - pk0
    d1 = fc - pk1
    valid &= (d0 >= 0) & (d1 >= 0)
    valid &= (jnp.mod(d0, 3) == 0) & (jnp.mod(d1, 3) == 0)
    j0 = d0 // 3
    j1 = d1 // 3
    valid &= (j0 < _NF) & (j1 < _NF)
    cell_ref[0] = jnp.where(valid, j0 * _NF + j1, _DUMP)


def _scatter_body(cell_hbm, rows_hbm, zeros_hbm, out_hbm, idx_v, rows_v, grid_sh):
    c = lax.axis_index("c")
    s = lax.axis_index("s")
    bl = s // 4
    q = s % 4
    b = c * 4 + bl

    pltpu.sync_copy(zeros_hbm.at[pl.ds(q * _QR, _QR)],
                    grid_sh.at[bl].at[pl.ds(q * _QR, _QR)])
    plsc.subcore_barrier()

    def body(j, carry):
        r = q * _CPT + j
        pltpu.sync_copy(cell_hbm.at[b, r], idx_v)
        pltpu.sync_copy(rows_hbm.at[b, pl.ds(r * _CH, _CH)], rows_v)
        pltpu.sync_copy(rows_v, grid_sh.at[bl].at[idx_v], add=True)
        return carry

    lax.fori_loop(0, _CPT, body, 0)
    plsc.subcore_barrier()
    pltpu.sync_copy(grid_sh.at[bl].at[pl.ds(q * _QR, _QR)],
                    out_hbm.at[b].at[pl.ds(q * _QR, _QR)])


@jax.jit
def kernel(pc1, features):
    B, _, N = pc1.shape

    cells = pl.pallas_call(
        _cells_body,
        grid=(B,),
        in_specs=[pl.BlockSpec((1, 3, N), lambda i: (i, 0, 0))],
        out_specs=pl.BlockSpec((1, 1, N), lambda i: (i, 0, 0)),
        out_shape=jax.ShapeDtypeStruct((B, 1, N), jnp.int32),
        compiler_params=pltpu.CompilerParams(vmem_limit_bytes=100 * 1024 * 1024),
    )(pc1)[:, 0, :]

    cell3 = jnp.pad(cells, ((0, 0), (0, _NPAD - N)), constant_values=_DUMP)
    cell3 = cell3.reshape(B, _NB, _CH)
    rows = jnp.concatenate(
        [jnp.swapaxes(features, 1, 2),
         jnp.zeros((B, N, 5), jnp.float32)], axis=-1)
    rows = jnp.pad(rows, ((0, 0), (0, _NPAD - N), (0, 0)))
    zeros = jnp.zeros((_GPAD, 8), jnp.float32)

    mesh = plsc.VectorSubcoreMesh(core_axis_name="c", subcore_axis_name="s")
    grid_out = pl.kernel(
        _scatter_body,
        out_type=jax.ShapeDtypeStruct((B, _GPAD, 8), jnp.float32),
        mesh=mesh,
        scratch_types=[
            pltpu.VMEM((_CH,), jnp.int32),
            pltpu.VMEM((_CH, 8), jnp.float32),
            pltpu.VMEM_SHARED((4, _GPAD, 8), jnp.float32),
        ],
        compiler_params=pltpu.CompilerParams(use_tc_tiling_on_sc=False),
    )(cell3, rows, zeros)

    out = grid_out[:, :_NCELL, :3].reshape(B, _NF, _NF, 3)
    return (out, out)


# speedup vs baseline: 50.7449x; 4.5635x over previous
import functools

import jax
import jax.numpy as jnp
from jax import lax
from jax.experimental import pallas as pl
from jax.experimental.pallas import tpu as pltpu
from jax.experimental.pallas import tpu_sc as plsc

_S = 512
_NF = _S // 3
_NCELL = _NF * _NF
_DUMP = _NCELL
_GPAD = 28928
_QR = _GPAD // 4
_CH = 128
_NB = 784
_CPT = _NB // 4
_NPAD = _NB * _CH


def _cells_body(pc_ref, cell_ref):
    pcm = pc_ref[0]
    rr = lax.broadcasted_iota(jnp.int32, (3, 3), 0)
    cc = lax.broadcasted_iota(jnp.int32, (3, 3), 1)
    eye = (rr == cc).astype(jnp.float32)
    emat = (3.0 * eye - 1.0) / jnp.sqrt(jnp.float32(6.0))
    ele = jnp.dot(emat, pcm, preferred_element_type=jnp.float32)
    e0 = ele[0:1, :]
    e1 = ele[1:2, :]
    e2 = ele[2:3, :]

    g0 = jnp.round(e0 / 3.0)
    g1 = jnp.round(e1 / 3.0)
    g2 = jnp.round(e2 / 3.0)
    x0 = e0 - g0 * 3.0
    x1 = e1 - g1 * 3.0
    x2 = e2 - g2 * 3.0

    i32 = jnp.int32
    r0 = (x1 > x0).astype(i32) + (x2 > x0).astype(i32)
    r1 = (x0 >= x1).astype(i32) + (x2 > x1).astype(i32)
    r2 = (x0 >= x2).astype(i32) + (x1 >= x2).astype(i32)

    rs = g0 + g1 + g2
    gt = rs > 0.0
    lt = rs < 0.0
    sign = jnp.where(gt, jnp.float32(-1.0), 0.0) + jnp.where(lt, jnp.float32(1.0), 0.0)

    def adjust(rk, g):
        rkf = rk.astype(jnp.float32)
        cond = ((rkf >= 3.0 - rs) & gt) | ((rkf < -rs) & lt)
        adj = sign * cond.astype(jnp.float32)
        g_new = g + adj
        rk_new = rk + (3.0 * adj).astype(i32) + rs.astype(i32)
        return rk_new, g_new

    r0, g0 = adjust(r0, g0)
    r1, g1 = adjust(r1, g1)
    r2, g2 = adjust(r2, g2)

    c0 = (g0 * 3.0).astype(i32)
    c1 = (g1 * 3.0).astype(i32)
    rc0 = jnp.clip(r0, 0, 2)
    rc1 = jnp.clip(r1, 0, 2)

    o0 = jnp.min(c0 - rc0)
    o1 = jnp.min(c1 - rc1)
    pk0 = jnp.mod(-o0, 3)
    pk1 = jnp.mod(-o1, 3)

    u0 = c0 - o0
    u1 = c1 - o1
    f = u0 * _S + u1
    valid = (f >= 0) & (f < _S * _S)
    fr = f >> 9
    fc = f & (_S - 1)
    d0 = fr - pk0
    d1 = fc - pk1
    valid &= (d0 >= 0) & (d1 >= 0)
    valid &= (jnp.mod(d0, 3) == 0) & (jnp.mod(d1, 3) == 0)
    j0 = d0 // 3
    j1 = d1 // 3
    valid &= (j0 < _NF) & (j1 < _NF)
    cell_ref[0] = jnp.where(valid, j0 * _NF + j1, _DUMP)


_PPT = 25000
_NCHUNK = _PPT // _CH
_TAIL = _PPT - _NCHUNK * _CH


def _scatter_body(cell_hbm, feat_hbm, zeros_hbm, out_hbm,
                  idx_v, st_v, rows_v, idx_t, st_t, rows_t, grid_sh):
    c = lax.axis_index("c")
    s = lax.axis_index("s")
    bl = s // 4
    q = s % 4
    b = c * 4 + bl
    base = q * _PPT

    pltpu.sync_copy(zeros_hbm.at[pl.ds(q * _QR, _QR)],
                    grid_sh.at[bl].at[pl.ds(q * _QR, _QR)])
    pltpu.sync_copy(zeros_hbm.at[pl.ds(0, _CH)], rows_v)
    pltpu.sync_copy(zeros_hbm.at[pl.ds(0, _TAIL)], rows_t)
    plsc.subcore_barrier()

    lanes = lax.iota(jnp.int32, 16)

    def interleave(st_ref, rows_ref, ngroups, limit):
        for ch in range(3):
            cidx = jnp.full((16,), ch, jnp.int32)
            for g in range(ngroups):
                vals = st_ref[ch, pl.ds(g * 16, 16)]
                ridx = g * 16 + lanes
                mask = None if limit is None else ridx < limit
                plsc.store_scatter(rows_ref, [ridx, cidx], vals, mask=mask)

    def body(j, carry):
        start = base + j * _CH
        pltpu.sync_copy(cell_hbm.at[b, pl.ds(start, _CH)], idx_v)
        pltpu.sync_copy(feat_hbm.at[b, :, pl.ds(start, _CH)], st_v)
        interleave(st_v, rows_v, _CH // 16, None)
        pltpu.sync_copy(rows_v, grid_sh.at[bl].at[idx_v], add=True)
        return carry

    lax.fori_loop(0, _NCHUNK, body, 0)

    tstart = base + _NCHUNK * _CH
    pltpu.sync_copy(cell_hbm.at[b, pl.ds(tstart, _TAIL)], idx_t)
    pltpu.sync_copy(feat_hbm.at[b, :, pl.ds(tstart, _TAIL)],
                    st_t.at[:, pl.ds(0, _TAIL)])
    interleave(st_t, rows_t, 3, _TAIL)
    pltpu.sync_copy(rows_t, grid_sh.at[bl].at[idx_t], add=True)

    plsc.subcore_barrier()
    pltpu.sync_copy(grid_sh.at[bl].at[pl.ds(q * _QR, _QR)],
                    out_hbm.at[b].at[pl.ds(q * _QR, _QR)])


@jax.jit
def kernel(pc1, features):
    B, _, N = pc1.shape

    cells = pl.pallas_call(
        _cells_body,
        grid=(B,),
        in_specs=[pl.BlockSpec((1, 3, N), lambda i: (i, 0, 0))],
        out_specs=pl.BlockSpec((1, 1, N), lambda i: (i, 0, 0)),
        out_shape=jax.ShapeDtypeStruct((B, 1, N), jnp.int32),
        compiler_params=pltpu.CompilerParams(vmem_limit_bytes=100 * 1024 * 1024),
    )(pc1)[:, 0, :]

    zeros = jnp.zeros((_GPAD, 8), jnp.float32)

    mesh = plsc.VectorSubcoreMesh(core_axis_name="c", subcore_axis_name="s")
    grid_out = pl.kernel(
        _scatter_body,
        out_type=jax.ShapeDtypeStruct((B, _GPAD, 8), jnp.float32),
        mesh=mesh,
        scratch_types=[
            pltpu.VMEM((_CH,), jnp.int32),
            pltpu.VMEM((3, _CH), jnp.float32),
            pltpu.VMEM((_CH, 8), jnp.float32),
            pltpu.VMEM((_TAIL,), jnp.int32),
            pltpu.VMEM((3, 48), jnp.float32),
            pltpu.VMEM((_TAIL, 8), jnp.float32),
            pltpu.VMEM_SHARED((4, _GPAD, 8), jnp.float32),
        ],
        compiler_params=pltpu.CompilerParams(use_tc_tiling_on_sc=False,
                                             needs_layout_passes=False),
    )(cells, features, zeros)

    out = grid_out[:, :_NCELL, :3].reshape(B, _NF, _NF, 3)
    return (out, out)


# speedup vs baseline: 67.1540x; 1.3234x over previous
import functools

import jax
import jax.numpy as jnp
from jax import lax
from jax.experimental import pallas as pl
from jax.experimental.pallas import tpu as pltpu
from jax.experimental.pallas import tpu_sc as plsc

_S = 512
_NF = _S // 3
_NCELL = _NF * _NF
_DUMP = _NCELL
_GPAD = 28928
_QR = _GPAD // 4
_CH = 128
_NB = 784
_CPT = _NB // 4
_NPAD = _NB * _CH


def _cells_body(pc_ref, cell_ref):
    pcm = pc_ref[0]
    rr = lax.broadcasted_iota(jnp.int32, (3, 3), 0)
    cc = lax.broadcasted_iota(jnp.int32, (3, 3), 1)
    eye = (rr == cc).astype(jnp.float32)
    emat = (3.0 * eye - 1.0) / jnp.sqrt(jnp.float32(6.0))
    ele = jnp.dot(emat, pcm, preferred_element_type=jnp.float32)
    n = pcm.shape[1]
    el3 = ele.reshape(3, 8, n // 8)
    e0 = el3[0]
    e1 = el3[1]
    e2 = el3[2]

    g0 = jnp.round(e0 / 3.0)
    g1 = jnp.round(e1 / 3.0)
    g2 = jnp.round(e2 / 3.0)
    x0 = e0 - g0 * 3.0
    x1 = e1 - g1 * 3.0
    x2 = e2 - g2 * 3.0

    i32 = jnp.int32
    r0 = (x1 > x0).astype(i32) + (x2 > x0).astype(i32)
    r1 = (x0 >= x1).astype(i32) + (x2 > x1).astype(i32)
    r2 = (x0 >= x2).astype(i32) + (x1 >= x2).astype(i32)

    rs = g0 + g1 + g2
    gt = rs > 0.0
    lt = rs < 0.0
    sign = jnp.where(gt, jnp.float32(-1.0), 0.0) + jnp.where(lt, jnp.float32(1.0), 0.0)

    def adjust(rk, g):
        rkf = rk.astype(jnp.float32)
        cond = ((rkf >= 3.0 - rs) & gt) | ((rkf < -rs) & lt)
        adj = sign * cond.astype(jnp.float32)
        g_new = g + adj
        rk_new = rk + (3.0 * adj).astype(i32) + rs.astype(i32)
        return rk_new, g_new

    r0, g0 = adjust(r0, g0)
    r1, g1 = adjust(r1, g1)
    r2, g2 = adjust(r2, g2)

    c0 = (g0 * 3.0).astype(i32)
    c1 = (g1 * 3.0).astype(i32)
    rc0 = jnp.clip(r0, 0, 2)
    rc1 = jnp.clip(r1, 0, 2)

    o0 = jnp.min(c0 - rc0)
    o1 = jnp.min(c1 - rc1)
    pk0 = jnp.mod(-o0, 3)
    pk1 = jnp.mod(-o1, 3)

    u0 = c0 - o0
    u1 = c1 - o1
    f = u0 * _S + u1
    valid = (f >= 0) & (f < _S * _S)
    fr = f >> 9
    fc = f & (_S - 1)
    d0 = fr - pk0
    d1 = fc - pk1
    valid &= (d0 >= 0) & (d1 >= 0)
    valid &= (jnp.mod(d0, 3) == 0) & (jnp.mod(d1, 3) == 0)
    j0 = d0 // 3
    j1 = d1 // 3
    valid &= (j0 < _NF) & (j1 < _NF)
    cell_ref[0] = jnp.where(valid, j0 * _NF + j1, _DUMP)


_PPT = 25000
_NCHUNK = _PPT // _CH
_TAIL = _PPT - _NCHUNK * _CH


def _scatter_body(cell_hbm, feat_hbm, zeros_hbm, out_hbm,
                  idx_v, st_v, rows_v, idx_t, st_t, rows_t, grid_sh):
    c = lax.axis_index("c")
    s = lax.axis_index("s")
    bl = s // 4
    q = s % 4
    b = c * 4 + bl
    base = q * _PPT

    pltpu.sync_copy(zeros_hbm.at[pl.ds(q * _QR, _QR)],
                    grid_sh.at[bl].at[pl.ds(q * _QR, _QR)])
    pltpu.sync_copy(zeros_hbm.at[pl.ds(0, _CH)], rows_v)
    pltpu.sync_copy(zeros_hbm.at[pl.ds(0, _TAIL)], rows_t)
    plsc.subcore_barrier()

    lanes = lax.iota(jnp.int32, 16)

    def interleave(st_ref, rows_ref, ngroups, limit):
        for ch in range(3):
            cidx = jnp.full((16,), ch, jnp.int32)
            for g in range(ngroups):
                vals = st_ref[ch, pl.ds(g * 16, 16)]
                ridx = g * 16 + lanes
                mask = None if limit is None else ridx < limit
                plsc.store_scatter(rows_ref, [ridx, cidx], vals, mask=mask)

    def body(j, carry):
        start = base + j * _CH
        pltpu.sync_copy(cell_hbm.at[b, pl.ds(start, _CH)], idx_v)
        pltpu.sync_copy(feat_hbm.at[b, :, pl.ds(start, _CH)], st_v)
        interleave(st_v, rows_v, _CH // 16, None)
        pltpu.sync_copy(rows_v, grid_sh.at[bl].at[idx_v], add=True)
        return carry

    lax.fori_loop(0, _NCHUNK, body, 0)

    tstart = base + _NCHUNK * _CH
    pltpu.sync_copy(cell_hbm.at[b, pl.ds(tstart, _TAIL)], idx_t)
    pltpu.sync_copy(feat_hbm.at[b, :, pl.ds(tstart, _TAIL)],
                    st_t.at[:, pl.ds(0, _TAIL)])
    interleave(st_t, rows_t, 3, _TAIL)
    pltpu.sync_copy(rows_t, grid_sh.at[bl].at[idx_t], add=True)

    plsc.subcore_barrier()
    pltpu.sync_copy(grid_sh.at[bl].at[pl.ds(q * _QR, _QR)],
                    out_hbm.at[b].at[pl.ds(q * _QR, _QR)])


@jax.jit
def kernel(pc1, features):
    B, _, N = pc1.shape

    cells = pl.pallas_call(
        _cells_body,
        grid=(B,),
        in_specs=[pl.BlockSpec((1, 3, N), lambda i: (i, 0, 0))],
        out_specs=pl.BlockSpec((1, 8, N // 8), lambda i: (i, 0, 0)),
        out_shape=jax.ShapeDtypeStruct((B, 8, N // 8), jnp.int32),
        compiler_params=pltpu.CompilerParams(vmem_limit_bytes=100 * 1024 * 1024),
    )(pc1).reshape(B, N)

    zeros = jnp.zeros((_GPAD, 8), jnp.float32)

    mesh = plsc.VectorSubcoreMesh(core_axis_name="c", subcore_axis_name="s")
    grid_out = pl.kernel(
        _scatter_body,
        out_type=jax.ShapeDtypeStruct((B, _GPAD, 8), jnp.float32),
        mesh=mesh,
        scratch_types=[
            pltpu.VMEM((_CH,), jnp.int32),
            pltpu.VMEM((3, _CH), jnp.float32),
            pltpu.VMEM((_CH, 8), jnp.float32),
            pltpu.VMEM((_TAIL,), jnp.int32),
            pltpu.VMEM((3, 48), jnp.float32),
            pltpu.VMEM((_TAIL, 8), jnp.float32),
            pltpu.VMEM_SHARED((4, _GPAD, 8), jnp.float32),
        ],
        compiler_params=pltpu.CompilerParams(use_tc_tiling_on_sc=False,
                                             needs_layout_passes=False),
    )(cells, features, zeros)

    out = grid_out[:, :_NCELL, :3].reshape(B, _NF, _NF, 3)
    return (out, out)
